# baseline (speedup 1.0000x reference)
import functools

import jax
import jax.numpy as jnp
from jax import lax
from jax.experimental import pallas as pl
from jax.experimental.pallas import tpu as pltpu
from jax.experimental.pallas import tpu_sc as plsc

N = 10000
F_IN = 128
HID = 128
C = 64
E = 320000

N_ACC = 10240
NTILES = 32
EPT = 10240
E_PAD = NTILES * EPT
CH = 128
NCH = EPT // CH
RPT = N_ACC // 16
DEG_W = 16
NB = 4
FAST_CID = 0
X_FAST = 160
X_SLOW = 0

_MESH = dict(core_axis_name="c", subcore_axis_name="s")
_SC_PARAMS = pltpu.CompilerParams(use_tc_tiling_on_sc=False)


def _deg_call(dst16):
  mesh = plsc.VectorSubcoreMesh(**_MESH)
  NV = EPT // 16
  NZ = N_ACC // 16

  @functools.partial(
      pl.kernel,
      mesh=mesh,
      out_type=jax.ShapeDtypeStruct((NTILES, N_ACC), jnp.float32),
      scratch_types=[
          pltpu.VMEM((NV, 16), jnp.int32),
          pltpu.VMEM((N_ACC,), jnp.float32),
      ],
      compiler_params=pltpu.CompilerParams(
          use_tc_tiling_on_sc=False, needs_layout_passes=False),
  )
  def k(dst_hbm, out_hbm, dst_a, deg_loc):
    cid = lax.axis_index("c")
    sid = lax.axis_index("s")
    wid = sid * 2 + cid
    zero_v = jnp.zeros((16,), jnp.float32)
    one_v = jnp.full((16,), 1.0, jnp.float32)

    def zbody(j, carry):
      deg_loc[pl.ds(j * 16, 16)] = zero_v
      return carry

    lax.fori_loop(0, NZ, zbody, 0)
    pltpu.sync_copy(dst_hbm.at[pl.ds(wid * NV, NV)], dst_a)

    def body(j, carry):
      plsc.addupdate_scatter(deg_loc, [dst_a[j]], one_v)
      return carry

    lax.fori_loop(0, NV, body, 0)
    pltpu.sync_copy(deg_loc, out_hbm.at[wid])

  return k(dst16)


def _seg_sum_call(F, stage):
  mesh = plsc.VectorSubcoreMesh(**_MESH)

  @functools.partial(
      pl.kernel,
      mesh=mesh,
      out_type=jax.ShapeDtypeStruct((2, N_ACC, F), jnp.float32),
      scratch_types=[
          pltpu.VMEM((X_FAST, CH), jnp.int32),
          pltpu.VMEM((X_FAST, CH), jnp.int32),
          [pltpu.VMEM((CH, F), jnp.float32)] * NB,
          [pltpu.SemaphoreType.DMA] * NB,
          [pltpu.SemaphoreType.DMA] * NB,
          pltpu.VMEM_SHARED((N_ACC, F), jnp.float32),
      ] + ([pltpu.VMEM_SHARED((N_ACC, F), jnp.float32)] if stage else []),
      compiler_params=_SC_PARAMS,
  )
  def k(y_hbm, src_hbm, dst_hbm, zeros_hbm, out_hbm, src_a, dst_a, rows, gsem, ssem, acc, *maybe_ysh):
    cid = lax.axis_index("c")
    sid = lax.axis_index("s")
    if stage:
      y_sh = maybe_ysh[0]
      pltpu.sync_copy(y_hbm.at[pl.ds(sid * RPT, RPT)], y_sh.at[pl.ds(sid * RPT, RPT)])
    else:
      y_sh = y_hbm
    pltpu.sync_copy(zeros_hbm, rows[0])
    for j in range(RPT // CH):
      pltpu.sync_copy(rows[0], acc.at[pl.ds(sid * RPT + j * CH, CH)])

    def run(base, xc):
      pltpu.sync_copy(src_hbm.at[pl.ds(base, xc)], src_a.at[pl.ds(0, xc)])
      pltpu.sync_copy(dst_hbm.at[pl.ds(base, xc)], dst_a.at[pl.ds(0, xc)])

      def body(i, carry):
        for kb in range(NB):
          @pl.when(i > 0)
          def _():
            pltpu.make_async_copy(rows[kb], acc.at[dst_a.at[0]], ssem[kb]).wait()
          pltpu.async_copy(y_sh.at[src_a.at[NB * i + kb]], rows[kb], gsem[kb])
        for kb in range(NB):
          pltpu.make_async_copy(y_hbm.at[pl.ds(0, CH)], rows[kb], gsem[kb]).wait()
          pltpu.async_copy(rows[kb], acc.at[dst_a.at[NB * i + kb]], ssem[kb], add=True)
        return carry

      lax.fori_loop(0, xc // NB, body, 0)
      for kb in range(NB):
        pltpu.make_async_copy(rows[kb], acc.at[dst_a.at[0]], ssem[kb]).wait()

    @pl.when(cid == FAST_CID)
    def _():
      run(sid * X_FAST, X_FAST)

    if X_SLOW > 0:
      @pl.when(cid == 1 - FAST_CID)
      def _():
        run(16 * X_FAST + sid * X_SLOW, X_SLOW)

    plsc.subcore_barrier()
    for j in range(RPT // CH):
      r = sid * RPT + j * CH
      pltpu.sync_copy(acc.at[pl.ds(r, CH)], rows[j % NB])
      pltpu.sync_copy(rows[j % NB], out_hbm.at[cid, pl.ds(r, CH)])

  return k


_seg_sum_64 = _seg_sum_call(C, stage=False)

RB = 256
GRID = N_ACC // RB


def _tc1_call(x_p, degp, W1):
  def body(x_ref, d_ref, w_ref, ya_ref, yb_ref, dinv_ref):
    d = jnp.sum(d_ref[...], axis=0).reshape(RB, 1) + 1.0
    dinv = lax.rsqrt(d)
    xw = jnp.dot(x_ref[...], w_ref[...], preferred_element_type=jnp.float32)
    y = xw * dinv
    ya_ref[...] = y[:, :C]
    yb_ref[...] = y[:, C:]
    dinv_ref[...] = dinv

  return pl.pallas_call(
      body,
      grid=(GRID,),
      in_specs=[
          pl.BlockSpec((RB, F_IN), lambda i: (i, 0)),
          pl.BlockSpec((NTILES, RB), lambda i: (0, i)),
          pl.BlockSpec((F_IN, HID), lambda i: (0, 0)),
      ],
      out_specs=[
          pl.BlockSpec((RB, C), lambda i: (i, 0)),
          pl.BlockSpec((RB, C), lambda i: (i, 0)),
          pl.BlockSpec((RB, 1), lambda i: (i, 0)),
      ],
      out_shape=[
          jax.ShapeDtypeStruct((N_ACC, C), jnp.float32),
          jax.ShapeDtypeStruct((N_ACC, C), jnp.float32),
          jax.ShapeDtypeStruct((N_ACC, 1), jnp.float32),
      ],
  )(x_p, degp, W1)


def _tc2_call(pa, pb, ya, yb, dinv, b, W):
  def body(pa_ref, pb_ref, ya_ref, yb_ref, dinv_ref, b_ref, w_ref, o_ref):
    dinv = dinv_ref[...]
    ta = dinv * (pa_ref[0] + pa_ref[1] + ya_ref[...]) + b_ref[:, :C]
    tb = dinv * (pb_ref[0] + pb_ref[1] + yb_ref[...]) + b_ref[:, C:]
    h = jnp.maximum(jnp.concatenate([ta, tb], axis=1), 0.0)
    o_ref[...] = jnp.dot(h, w_ref[...], preferred_element_type=jnp.float32) * dinv

  return pl.pallas_call(
      body,
      grid=(GRID,),
      in_specs=[
          pl.BlockSpec((2, RB, C), lambda i: (0, i, 0)),
          pl.BlockSpec((2, RB, C), lambda i: (0, i, 0)),
          pl.BlockSpec((RB, C), lambda i: (i, 0)),
          pl.BlockSpec((RB, C), lambda i: (i, 0)),
          pl.BlockSpec((RB, 1), lambda i: (i, 0)),
          pl.BlockSpec((1, HID), lambda i: (0, 0)),
          pl.BlockSpec((HID, C), lambda i: (0, 0)),
      ],
      out_specs=pl.BlockSpec((RB, C), lambda i: (i, 0)),
      out_shape=jax.ShapeDtypeStruct((N_ACC, C), jnp.float32),
  )(pa, pb, ya, yb, dinv, b, W)


def _tc_mid_call(p, y, dinv, b, W, F_in, F_out, act):
  def body(p_ref, y_ref, dinv_ref, b_ref, w_ref, o_ref):
    dinv = dinv_ref[...]
    t = dinv * (p_ref[0] + p_ref[1] + y_ref[...]) + b_ref[...]
    if act == "relu":
      h = jnp.maximum(t, 0.0)
    else:
      m = jnp.max(t, axis=1, keepdims=True)
      ex = jnp.exp(t - m)
      h = ex / jnp.sum(ex, axis=1, keepdims=True)
    o_ref[...] = jnp.dot(h, w_ref[...], preferred_element_type=jnp.float32) * dinv

  return pl.pallas_call(
      body,
      grid=(GRID,),
      in_specs=[
          pl.BlockSpec((2, RB, F_in), lambda i: (0, i, 0)),
          pl.BlockSpec((RB, F_in), lambda i: (i, 0)),
          pl.BlockSpec((RB, 1), lambda i: (i, 0)),
          pl.BlockSpec((1, F_in), lambda i: (0, 0)),
          pl.BlockSpec((F_in, F_out), lambda i: (0, 0)),
      ],
      out_specs=pl.BlockSpec((RB, F_out), lambda i: (i, 0)),
      out_shape=jax.ShapeDtypeStruct((N_ACC, F_out), jnp.float32),
  )(p, y, dinv, b, W)


def _tc_out_call(p, y, dinv, b):
  def body(p_ref, y_ref, dinv_ref, b_ref, o_ref):
    o_ref[...] = dinv_ref[...] * (p_ref[0] + p_ref[1] + y_ref[...]) + b_ref[...]

  return pl.pallas_call(
      body,
      grid=(GRID,),
      in_specs=[
          pl.BlockSpec((2, RB, C), lambda i: (0, i, 0)),
          pl.BlockSpec((RB, C), lambda i: (i, 0)),
          pl.BlockSpec((RB, 1), lambda i: (i, 0)),
          pl.BlockSpec((1, C), lambda i: (0, 0)),
      ],
      out_specs=pl.BlockSpec((RB, C), lambda i: (i, 0)),
      out_shape=jax.ShapeDtypeStruct((N_ACC, C), jnp.float32),
  )(p, y, dinv, b)


def kernel(x, edge_index, W1, b1, W2, b2, W3, b3):
  src = edge_index[0]
  dst = edge_index[1]
  pad_e = E_PAD - E
  padv = jnp.full((pad_e,), N, jnp.int32)
  src_p = jnp.concatenate([src, padv]).reshape(E_PAD // CH, CH)
  dst_p = jnp.concatenate([dst, padv]).reshape(E_PAD // CH, CH)
  x_p = jnp.pad(x, ((0, N_ACC - N), (0, 0)))

  zeros64 = jnp.zeros((CH, C), jnp.float32)

  degp = _deg_call(dst_p.reshape(E_PAD // 16, 16))
  y1a, y1b, dinv = _tc1_call(x_p, degp, W1)
  p1a = _seg_sum_64(y1a, src_p, dst_p, zeros64)
  p1b = _seg_sum_64(y1b, src_p, dst_p, zeros64)
  y2 = _tc2_call(p1a, p1b, y1a, y1b, dinv, b1.reshape(1, -1), W2)
  p2 = _seg_sum_64(y2, src_p, dst_p, zeros64)
  y3 = _tc_mid_call(p2, y2, dinv, b2.reshape(1, -1), W3, C, C, "softmax")
  p3 = _seg_sum_64(y3, src_p, dst_p, zeros64)
  out = _tc_out_call(p3, y3, dinv, b3.reshape(1, -1))
  return out[:N]


# speedup vs baseline: 1.2457x; 1.2457x over previous
import functools

import jax
import jax.numpy as jnp
from jax import lax
from jax.experimental import pallas as pl
from jax.experimental.pallas import tpu as pltpu
from jax.experimental.pallas import tpu_sc as plsc

N = 10000
F_IN = 128
HID = 128
C = 64
E = 320000

N_ACC = 10240
NTILES = 32
EPT = 10240
E_PAD = NTILES * EPT
CH = 128
NCH = EPT // CH
RPT = N_ACC // 16
DEG_W = 16
NB = 4
FAST_CID = 0
X_FAST = 144
X_SLOW = 16

_MESH = dict(core_axis_name="c", subcore_axis_name="s")
_SC_PARAMS = pltpu.CompilerParams(use_tc_tiling_on_sc=False)


def _deg_call(dst16):
  mesh = plsc.VectorSubcoreMesh(**_MESH)
  NV = EPT // 16
  NZ = N_ACC // 16

  @functools.partial(
      pl.kernel,
      mesh=mesh,
      out_type=jax.ShapeDtypeStruct((NTILES, N_ACC), jnp.float32),
      scratch_types=[
          pltpu.VMEM((NV, 16), jnp.int32),
          pltpu.VMEM((N_ACC,), jnp.float32),
      ],
      compiler_params=pltpu.CompilerParams(
          use_tc_tiling_on_sc=False, needs_layout_passes=False),
  )
  def k(dst_hbm, out_hbm, dst_a, deg_loc):
    cid = lax.axis_index("c")
    sid = lax.axis_index("s")
    wid = sid * 2 + cid
    zero_v = jnp.zeros((16,), jnp.float32)
    one_v = jnp.full((16,), 1.0, jnp.float32)

    def zbody(j, carry):
      deg_loc[pl.ds(j * 16, 16)] = zero_v
      return carry

    lax.fori_loop(0, NZ, zbody, 0)
    pltpu.sync_copy(dst_hbm.at[pl.ds(wid * NV, NV)], dst_a)

    def body(j, carry):
      plsc.addupdate_scatter(deg_loc, [dst_a[j]], one_v)
      return carry

    lax.fori_loop(0, NV, body, 0)
    pltpu.sync_copy(deg_loc, out_hbm.at[wid])

  return k(dst16)


def _seg_sum_call(F, stage):
  mesh = plsc.VectorSubcoreMesh(**_MESH)

  @functools.partial(
      pl.kernel,
      mesh=mesh,
      out_type=jax.ShapeDtypeStruct((2, N_ACC, F), jnp.float32),
      scratch_types=[
          pltpu.VMEM((X_FAST, CH), jnp.int32),
          pltpu.VMEM((X_FAST, CH), jnp.int32),
          [pltpu.VMEM((CH, F), jnp.float32)] * NB,
          [pltpu.SemaphoreType.DMA] * NB,
          [pltpu.SemaphoreType.DMA] * NB,
          pltpu.VMEM_SHARED((N_ACC, F), jnp.float32),
      ] + ([pltpu.VMEM_SHARED((N_ACC, F), jnp.float32)] if stage else []),
      compiler_params=_SC_PARAMS,
  )
  def k(y_hbm, src_hbm, dst_hbm, zeros_hbm, out_hbm, src_a, dst_a, rows, gsem, ssem, acc, *maybe_ysh):
    cid = lax.axis_index("c")
    sid = lax.axis_index("s")
    if stage:
      y_sh = maybe_ysh[0]
      pltpu.sync_copy(y_hbm.at[pl.ds(sid * RPT, RPT)], y_sh.at[pl.ds(sid * RPT, RPT)])
    else:
      y_sh = y_hbm
    pltpu.sync_copy(zeros_hbm, rows[0])
    for j in range(RPT // CH):
      pltpu.sync_copy(rows[0], acc.at[pl.ds(sid * RPT + j * CH, CH)])

    def run(base, xc):
      pltpu.sync_copy(src_hbm.at[pl.ds(base, xc)], src_a.at[pl.ds(0, xc)])
      pltpu.sync_copy(dst_hbm.at[pl.ds(base, xc)], dst_a.at[pl.ds(0, xc)])

      def body(i, carry):
        for kb in range(NB):
          @pl.when(i > 0)
          def _():
            pltpu.make_async_copy(rows[kb], acc.at[dst_a.at[0]], ssem[kb]).wait()
          pltpu.async_copy(y_sh.at[src_a.at[NB * i + kb]], rows[kb], gsem[kb])
        for kb in range(NB):
          pltpu.make_async_copy(y_hbm.at[pl.ds(0, CH)], rows[kb], gsem[kb]).wait()
          pltpu.async_copy(rows[kb], acc.at[dst_a.at[NB * i + kb]], ssem[kb], add=True)
        return carry

      lax.fori_loop(0, xc // NB, body, 0)
      for kb in range(NB):
        pltpu.make_async_copy(rows[kb], acc.at[dst_a.at[0]], ssem[kb]).wait()

    @pl.when(cid == FAST_CID)
    def _():
      run(sid * X_FAST, X_FAST)

    if X_SLOW > 0:
      @pl.when(cid == 1 - FAST_CID)
      def _():
        run(16 * X_FAST + sid * X_SLOW, X_SLOW)

    plsc.subcore_barrier()
    for j in range(RPT // CH):
      r = sid * RPT + j * CH
      pltpu.sync_copy(acc.at[pl.ds(r, CH)], rows[j % NB])
      pltpu.sync_copy(rows[j % NB], out_hbm.at[cid, pl.ds(r, CH)])

  return k


_seg_sum_64 = _seg_sum_call(C, stage=False)

RB = 256
GRID = N_ACC // RB


def _tc1_call(x_p, degp, W1):
  def body(x_ref, d_ref, w_ref, ya_ref, yb_ref, dinv_ref):
    d = jnp.sum(d_ref[...], axis=0).reshape(RB, 1) + 1.0
    dinv = lax.rsqrt(d)
    xw = jnp.dot(x_ref[...], w_ref[...], preferred_element_type=jnp.float32)
    y = xw * dinv
    ya_ref[...] = y[:, :C]
    yb_ref[...] = y[:, C:]
    dinv_ref[...] = dinv

  return pl.pallas_call(
      body,
      grid=(GRID,),
      in_specs=[
          pl.BlockSpec((RB, F_IN), lambda i: (i, 0)),
          pl.BlockSpec((NTILES, RB), lambda i: (0, i)),
          pl.BlockSpec((F_IN, HID), lambda i: (0, 0)),
      ],
      out_specs=[
          pl.BlockSpec((RB, C), lambda i: (i, 0)),
          pl.BlockSpec((RB, C), lambda i: (i, 0)),
          pl.BlockSpec((RB, 1), lambda i: (i, 0)),
      ],
      out_shape=[
          jax.ShapeDtypeStruct((N_ACC, C), jnp.float32),
          jax.ShapeDtypeStruct((N_ACC, C), jnp.float32),
          jax.ShapeDtypeStruct((N_ACC, 1), jnp.float32),
      ],
  )(x_p, degp, W1)


def _tc2_call(pa, pb, ya, yb, dinv, b, W):
  def body(pa_ref, pb_ref, ya_ref, yb_ref, dinv_ref, b_ref, w_ref, o_ref):
    dinv = dinv_ref[...]
    ta = dinv * (pa_ref[0] + pa_ref[1] + ya_ref[...]) + b_ref[:, :C]
    tb = dinv * (pb_ref[0] + pb_ref[1] + yb_ref[...]) + b_ref[:, C:]
    h = jnp.maximum(jnp.concatenate([ta, tb], axis=1), 0.0)
    o_ref[...] = jnp.dot(h, w_ref[...], preferred_element_type=jnp.float32) * dinv

  return pl.pallas_call(
      body,
      grid=(GRID,),
      in_specs=[
          pl.BlockSpec((2, RB, C), lambda i: (0, i, 0)),
          pl.BlockSpec((2, RB, C), lambda i: (0, i, 0)),
          pl.BlockSpec((RB, C), lambda i: (i, 0)),
          pl.BlockSpec((RB, C), lambda i: (i, 0)),
          pl.BlockSpec((RB, 1), lambda i: (i, 0)),
          pl.BlockSpec((1, HID), lambda i: (0, 0)),
          pl.BlockSpec((HID, C), lambda i: (0, 0)),
      ],
      out_specs=pl.BlockSpec((RB, C), lambda i: (i, 0)),
      out_shape=jax.ShapeDtypeStruct((N_ACC, C), jnp.float32),
  )(pa, pb, ya, yb, dinv, b, W)


def _tc_mid_call(p, y, dinv, b, W, F_in, F_out, act):
  def body(p_ref, y_ref, dinv_ref, b_ref, w_ref, o_ref):
    dinv = dinv_ref[...]
    t = dinv * (p_ref[0] + p_ref[1] + y_ref[...]) + b_ref[...]
    if act == "relu":
      h = jnp.maximum(t, 0.0)
    else:
      m = jnp.max(t, axis=1, keepdims=True)
      ex = jnp.exp(t - m)
      h = ex / jnp.sum(ex, axis=1, keepdims=True)
    o_ref[...] = jnp.dot(h, w_ref[...], preferred_element_type=jnp.float32) * dinv

  return pl.pallas_call(
      body,
      grid=(GRID,),
      in_specs=[
          pl.BlockSpec((2, RB, F_in), lambda i: (0, i, 0)),
          pl.BlockSpec((RB, F_in), lambda i: (i, 0)),
          pl.BlockSpec((RB, 1), lambda i: (i, 0)),
          pl.BlockSpec((1, F_in), lambda i: (0, 0)),
          pl.BlockSpec((F_in, F_out), lambda i: (0, 0)),
      ],
      out_specs=pl.BlockSpec((RB, F_out), lambda i: (i, 0)),
      out_shape=jax.ShapeDtypeStruct((N_ACC, F_out), jnp.float32),
  )(p, y, dinv, b, W)


def _tc_out_call(p, y, dinv, b):
  def body(p_ref, y_ref, dinv_ref, b_ref, o_ref):
    o_ref[...] = dinv_ref[...] * (p_ref[0] + p_ref[1] + y_ref[...]) + b_ref[...]

  return pl.pallas_call(
      body,
      grid=(GRID,),
      in_specs=[
          pl.BlockSpec((2, RB, C), lambda i: (0, i, 0)),
          pl.BlockSpec((RB, C), lambda i: (i, 0)),
          pl.BlockSpec((RB, 1), lambda i: (i, 0)),
          pl.BlockSpec((1, C), lambda i: (0, 0)),
      ],
      out_specs=pl.BlockSpec((RB, C), lambda i: (i, 0)),
      out_shape=jax.ShapeDtypeStruct((N_ACC, C), jnp.float32),
  )(p, y, dinv, b)


def kernel(x, edge_index, W1, b1, W2, b2, W3, b3):
  src = edge_index[0]
  dst = edge_index[1]
  pad_e = E_PAD - E
  padv = jnp.full((pad_e,), N, jnp.int32)
  src_p = jnp.concatenate([src, padv]).reshape(E_PAD // CH, CH)
  dst_p = jnp.concatenate([dst, padv]).reshape(E_PAD // CH, CH)
  x_p = jnp.pad(x, ((0, N_ACC - N), (0, 0)))

  zeros64 = jnp.zeros((CH, C), jnp.float32)

  degp = _deg_call(dst_p.reshape(E_PAD // 16, 16))
  y1a, y1b, dinv = _tc1_call(x_p, degp, W1)
  p1a = _seg_sum_64(y1a, src_p, dst_p, zeros64)
  p1b = _seg_sum_64(y1b, src_p, dst_p, zeros64)
  y2 = _tc2_call(p1a, p1b, y1a, y1b, dinv, b1.reshape(1, -1), W2)
  p2 = _seg_sum_64(y2, src_p, dst_p, zeros64)
  y3 = _tc_mid_call(p2, y2, dinv, b2.reshape(1, -1), W3, C, C, "softmax")
  p3 = _seg_sum_64(y3, src_p, dst_p, zeros64)
  out = _tc_out_call(p3, y3, dinv, b3.reshape(1, -1))
  return out[:N]


# speedup vs baseline: 1.2504x; 1.0038x over previous
import functools

import jax
import jax.numpy as jnp
from jax import lax
from jax.experimental import pallas as pl
from jax.experimental.pallas import tpu as pltpu
from jax.experimental.pallas import tpu_sc as plsc

N = 10000
F_IN = 128
HID = 128
C = 64
E = 320000

N_ACC = 10240
NTILES = 32
EPT = 10240
E_PAD = NTILES * EPT
CH = 128
NCH = EPT // CH
RPT = N_ACC // 16
DEG_W = 16
NB = 4
FAST_CID = 0
X_FAST = 152
X_SLOW = 8

_MESH = dict(core_axis_name="c", subcore_axis_name="s")
_SC_PARAMS = pltpu.CompilerParams(use_tc_tiling_on_sc=False)


def _deg_call(dst16):
  mesh = plsc.VectorSubcoreMesh(**_MESH)
  NV = EPT // 16
  NZ = N_ACC // 16

  @functools.partial(
      pl.kernel,
      mesh=mesh,
      out_type=jax.ShapeDtypeStruct((NTILES, N_ACC), jnp.float32),
      scratch_types=[
          pltpu.VMEM((NV, 16), jnp.int32),
          pltpu.VMEM((N_ACC,), jnp.float32),
      ],
      compiler_params=pltpu.CompilerParams(
          use_tc_tiling_on_sc=False, needs_layout_passes=False),
  )
  def k(dst_hbm, out_hbm, dst_a, deg_loc):
    cid = lax.axis_index("c")
    sid = lax.axis_index("s")
    wid = sid * 2 + cid
    zero_v = jnp.zeros((16,), jnp.float32)
    one_v = jnp.full((16,), 1.0, jnp.float32)

    def zbody(j, carry):
      deg_loc[pl.ds(j * 16, 16)] = zero_v
      return carry

    lax.fori_loop(0, NZ, zbody, 0)
    pltpu.sync_copy(dst_hbm.at[pl.ds(wid * NV, NV)], dst_a)

    def body(j, carry):
      plsc.addupdate_scatter(deg_loc, [dst_a[j]], one_v)
      return carry

    lax.fori_loop(0, NV, body, 0)
    pltpu.sync_copy(deg_loc, out_hbm.at[wid])

  return k(dst16)


def _seg_sum_call(F, stage):
  mesh = plsc.VectorSubcoreMesh(**_MESH)

  @functools.partial(
      pl.kernel,
      mesh=mesh,
      out_type=jax.ShapeDtypeStruct((2, N_ACC, F), jnp.float32),
      scratch_types=[
          pltpu.VMEM((X_FAST, CH), jnp.int32),
          pltpu.VMEM((X_FAST, CH), jnp.int32),
          [pltpu.VMEM((CH, F), jnp.float32)] * NB,
          [pltpu.SemaphoreType.DMA] * NB,
          [pltpu.SemaphoreType.DMA] * NB,
          pltpu.VMEM_SHARED((N_ACC, F), jnp.float32),
      ] + ([pltpu.VMEM_SHARED((N_ACC, F), jnp.float32)] if stage else []),
      compiler_params=_SC_PARAMS,
  )
  def k(y_hbm, src_hbm, dst_hbm, zeros_hbm, out_hbm, src_a, dst_a, rows, gsem, ssem, acc, *maybe_ysh):
    cid = lax.axis_index("c")
    sid = lax.axis_index("s")
    if stage:
      y_sh = maybe_ysh[0]
      pltpu.sync_copy(y_hbm.at[pl.ds(sid * RPT, RPT)], y_sh.at[pl.ds(sid * RPT, RPT)])
    else:
      y_sh = y_hbm
    pltpu.sync_copy(zeros_hbm, rows[0])
    for j in range(RPT // CH):
      pltpu.sync_copy(rows[0], acc.at[pl.ds(sid * RPT + j * CH, CH)])

    def run(base, xc):
      pltpu.sync_copy(src_hbm.at[pl.ds(base, xc)], src_a.at[pl.ds(0, xc)])
      pltpu.sync_copy(dst_hbm.at[pl.ds(base, xc)], dst_a.at[pl.ds(0, xc)])

      def body(i, carry):
        for kb in range(NB):
          @pl.when(i > 0)
          def _():
            pltpu.make_async_copy(rows[kb], acc.at[dst_a.at[0]], ssem[kb]).wait()
          pltpu.async_copy(y_sh.at[src_a.at[NB * i + kb]], rows[kb], gsem[kb])
        for kb in range(NB):
          pltpu.make_async_copy(y_hbm.at[pl.ds(0, CH)], rows[kb], gsem[kb]).wait()
          pltpu.async_copy(rows[kb], acc.at[dst_a.at[NB * i + kb]], ssem[kb], add=True)
        return carry

      lax.fori_loop(0, xc // NB, body, 0)
      for kb in range(NB):
        pltpu.make_async_copy(rows[kb], acc.at[dst_a.at[0]], ssem[kb]).wait()

    @pl.when(cid == FAST_CID)
    def _():
      run(sid * X_FAST, X_FAST)

    if X_SLOW > 0:
      @pl.when(cid == 1 - FAST_CID)
      def _():
        run(16 * X_FAST + sid * X_SLOW, X_SLOW)

    plsc.subcore_barrier()
    for j in range(RPT // CH):
      r = sid * RPT + j * CH
      pltpu.sync_copy(acc.at[pl.ds(r, CH)], rows[j % NB])
      pltpu.sync_copy(rows[j % NB], out_hbm.at[cid, pl.ds(r, CH)])

  return k


_seg_sum_64 = _seg_sum_call(C, stage=False)

RB = 256
GRID = N_ACC // RB


def _tc1_call(x_p, degp, W1):
  def body(x_ref, d_ref, w_ref, ya_ref, yb_ref, dinv_ref):
    d = jnp.sum(d_ref[...], axis=0).reshape(RB, 1) + 1.0
    dinv = lax.rsqrt(d)
    xw = jnp.dot(x_ref[...], w_ref[...], preferred_element_type=jnp.float32)
    y = xw * dinv
    ya_ref[...] = y[:, :C]
    yb_ref[...] = y[:, C:]
    dinv_ref[...] = dinv

  return pl.pallas_call(
      body,
      grid=(GRID,),
      in_specs=[
          pl.BlockSpec((RB, F_IN), lambda i: (i, 0)),
          pl.BlockSpec((NTILES, RB), lambda i: (0, i)),
          pl.BlockSpec((F_IN, HID), lambda i: (0, 0)),
      ],
      out_specs=[
          pl.BlockSpec((RB, C), lambda i: (i, 0)),
          pl.BlockSpec((RB, C), lambda i: (i, 0)),
          pl.BlockSpec((RB, 1), lambda i: (i, 0)),
      ],
      out_shape=[
          jax.ShapeDtypeStruct((N_ACC, C), jnp.float32),
          jax.ShapeDtypeStruct((N_ACC, C), jnp.float32),
          jax.ShapeDtypeStruct((N_ACC, 1), jnp.float32),
      ],
  )(x_p, degp, W1)


def _tc2_call(pa, pb, ya, yb, dinv, b, W):
  def body(pa_ref, pb_ref, ya_ref, yb_ref, dinv_ref, b_ref, w_ref, o_ref):
    dinv = dinv_ref[...]
    ta = dinv * (pa_ref[0] + pa_ref[1] + ya_ref[...]) + b_ref[:, :C]
    tb = dinv * (pb_ref[0] + pb_ref[1] + yb_ref[...]) + b_ref[:, C:]
    h = jnp.maximum(jnp.concatenate([ta, tb], axis=1), 0.0)
    o_ref[...] = jnp.dot(h, w_ref[...], preferred_element_type=jnp.float32) * dinv

  return pl.pallas_call(
      body,
      grid=(GRID,),
      in_specs=[
          pl.BlockSpec((2, RB, C), lambda i: (0, i, 0)),
          pl.BlockSpec((2, RB, C), lambda i: (0, i, 0)),
          pl.BlockSpec((RB, C), lambda i: (i, 0)),
          pl.BlockSpec((RB, C), lambda i: (i, 0)),
          pl.BlockSpec((RB, 1), lambda i: (i, 0)),
          pl.BlockSpec((1, HID), lambda i: (0, 0)),
          pl.BlockSpec((HID, C), lambda i: (0, 0)),
      ],
      out_specs=pl.BlockSpec((RB, C), lambda i: (i, 0)),
      out_shape=jax.ShapeDtypeStruct((N_ACC, C), jnp.float32),
  )(pa, pb, ya, yb, dinv, b, W)


def _tc_mid_call(p, y, dinv, b, W, F_in, F_out, act):
  def body(p_ref, y_ref, dinv_ref, b_ref, w_ref, o_ref):
    dinv = dinv_ref[...]
    t = dinv * (p_ref[0] + p_ref[1] + y_ref[...]) + b_ref[...]
    if act == "relu":
      h = jnp.maximum(t, 0.0)
    else:
      m = jnp.max(t, axis=1, keepdims=True)
      ex = jnp.exp(t - m)
      h = ex / jnp.sum(ex, axis=1, keepdims=True)
    o_ref[...] = jnp.dot(h, w_ref[...], preferred_element_type=jnp.float32) * dinv

  return pl.pallas_call(
      body,
      grid=(GRID,),
      in_specs=[
          pl.BlockSpec((2, RB, F_in), lambda i: (0, i, 0)),
          pl.BlockSpec((RB, F_in), lambda i: (i, 0)),
          pl.BlockSpec((RB, 1), lambda i: (i, 0)),
          pl.BlockSpec((1, F_in), lambda i: (0, 0)),
          pl.BlockSpec((F_in, F_out), lambda i: (0, 0)),
      ],
      out_specs=pl.BlockSpec((RB, F_out), lambda i: (i, 0)),
      out_shape=jax.ShapeDtypeStruct((N_ACC, F_out), jnp.float32),
  )(p, y, dinv, b, W)


def _tc_out_call(p, y, dinv, b):
  def body(p_ref, y_ref, dinv_ref, b_ref, o_ref):
    o_ref[...] = dinv_ref[...] * (p_ref[0] + p_ref[1] + y_ref[...]) + b_ref[...]

  return pl.pallas_call(
      body,
      grid=(GRID,),
      in_specs=[
          pl.BlockSpec((2, RB, C), lambda i: (0, i, 0)),
          pl.BlockSpec((RB, C), lambda i: (i, 0)),
          pl.BlockSpec((RB, 1), lambda i: (i, 0)),
          pl.BlockSpec((1, C), lambda i: (0, 0)),
      ],
      out_specs=pl.BlockSpec((RB, C), lambda i: (i, 0)),
      out_shape=jax.ShapeDtypeStruct((N_ACC, C), jnp.float32),
  )(p, y, dinv, b)


def kernel(x, edge_index, W1, b1, W2, b2, W3, b3):
  src = edge_index[0]
  dst = edge_index[1]
  pad_e = E_PAD - E
  padv = jnp.full((pad_e,), N, jnp.int32)
  src_p = jnp.concatenate([src, padv]).reshape(E_PAD // CH, CH)
  dst_p = jnp.concatenate([dst, padv]).reshape(E_PAD // CH, CH)
  x_p = jnp.pad(x, ((0, N_ACC - N), (0, 0)))

  zeros64 = jnp.zeros((CH, C), jnp.float32)

  degp = _deg_call(dst_p.reshape(E_PAD // 16, 16))
  y1a, y1b, dinv = _tc1_call(x_p, degp, W1)
  p1a = _seg_sum_64(y1a, src_p, dst_p, zeros64)
  p1b = _seg_sum_64(y1b, src_p, dst_p, zeros64)
  y2 = _tc2_call(p1a, p1b, y1a, y1b, dinv, b1.reshape(1, -1), W2)
  p2 = _seg_sum_64(y2, src_p, dst_p, zeros64)
  y3 = _tc_mid_call(p2, y2, dinv, b2.reshape(1, -1), W3, C, C, "softmax")
  p3 = _seg_sum_64(y3, src_p, dst_p, zeros64)
  out = _tc_out_call(p3, y3, dinv, b3.reshape(1, -1))
  return out[:N]
